# Initial kernel scaffold; baseline (speedup 1.0000x reference)
import jax
import jax.numpy as jnp
from jax.experimental import pallas as pl


def kernel(g, features, W, b):
    raise NotImplementedError("write your pallas kernel here")


# speedup vs baseline: 7.8263x; 7.8263x over previous
import functools

import jax
import jax.numpy as jnp
from jax import lax
from jax.experimental import pallas as pl
from jax.experimental.pallas import tpu as pltpu
from jax.experimental.pallas import tpu_sc as plsc

NC = 2
NS = 16
NW = NC * NS


def _deg_body(srcr, dstr, ones_h, zeros_h, out, sidx, didx, ones_v, zbuf,
              dbuf, acc_s, acc_d):
    npad = acc_s.shape[0]
    per_tile = npad // NS
    cid = lax.axis_index("c")
    sid = lax.axis_index("s")
    wid = cid * NS + sid
    nch = sidx.shape[0]

    pltpu.sync_copy(srcr.at[wid], sidx)
    pltpu.sync_copy(dstr.at[wid], didx)
    pltpu.sync_copy(ones_h, ones_v)
    pltpu.sync_copy(zeros_h, zbuf)

    base = sid * per_tile
    pltpu.sync_copy(zbuf, acc_s.at[pl.ds(base, per_tile)])
    pltpu.sync_copy(zbuf, acc_d.at[pl.ds(base, per_tile)])
    plsc.subcore_barrier()

    def body(j, carry):
        pltpu.sync_copy(ones_v, acc_s.at[sidx.at[j]], add=True)
        pltpu.sync_copy(ones_v, acc_d.at[didx.at[j]], add=True)
        return carry

    lax.fori_loop(0, nch, body, 0)
    plsc.subcore_barrier()

    pltpu.sync_copy(acc_s.at[pl.ds(base, per_tile)], dbuf)
    pltpu.sync_copy(dbuf, out.at[pl.ds(cid * 2 * npad + base, per_tile)])
    pltpu.sync_copy(acc_d.at[pl.ds(base, per_tile)], dbuf)
    pltpu.sync_copy(dbuf, out.at[pl.ds((cid * 2 + 1) * npad + base,
                                       per_tile)])


def _agg_body(ys, srcr, dstr, zeros_h, out, sidx, didx, rows, acc, sem):
    npad, d = acc.shape
    rows_per_tile = npad // NS
    chunk = rows.shape[0]
    ncopy = rows_per_tile // chunk
    cid = lax.axis_index("c")
    sid = lax.axis_index("s")
    wid = cid * NS + sid
    nch = sidx.shape[0]

    pltpu.sync_copy(srcr.at[wid], sidx)
    pltpu.sync_copy(dstr.at[wid], didx)

    pltpu.sync_copy(zeros_h, rows)
    rowbase = sid * rows_per_tile
    for k in range(ncopy):
        pltpu.sync_copy(rows, acc.at[pl.ds(rowbase + k * chunk, chunk)])
    plsc.subcore_barrier()

    def body(j, carry):
        pltpu.async_copy(ys.at[sidx.at[j]], rows, sem).wait()
        pltpu.sync_copy(rows, acc.at[didx.at[j]], add=True)
        return carry

    lax.fori_loop(0, nch, body, 0)
    plsc.subcore_barrier()

    for k in range(ncopy):
        sl = pl.ds(rowbase + k * chunk, chunk)
        pltpu.sync_copy(acc.at[sl], rows)
        pltpu.sync_copy(rows, out.at[cid, sl])


def _mm_body(x_ref, w_ref, p0_ref, p1_ref, o_ref):
    deg = p0_ref[...] + p1_ref[...]
    norm = lax.rsqrt(jnp.maximum(deg, 1.0))
    y = jnp.dot(x_ref[...], w_ref[...], preferred_element_type=jnp.float32)
    o_ref[...] = y * norm


def _fin_body(a0_ref, a1_ref, q0_ref, q1_ref, b_ref, o_ref):
    deg = q0_ref[...] + q1_ref[...]
    norm = lax.rsqrt(jnp.maximum(deg, 1.0))
    s = (a0_ref[...] + a1_ref[...]) * norm + b_ref[...]
    o_ref[...] = jnp.maximum(s, 0.0)


def kernel(g, features, W, b):
    n, d = features.shape
    e = g.shape[1]
    d_out = W.shape[1]

    chunk = 80
    assert e % (NW * chunk) == 0
    nch = e // (NW * chunk)
    npad = ((n + NS * 128 - 1) // (NS * 128)) * (NS * 128)
    deg_pt = npad // NS
    assert (npad // NS) % chunk == 0

    g32 = g.astype(jnp.int32)
    srcr = g32[0].reshape(NW, nch, chunk)
    dstr = g32[1].reshape(NW, nch, chunk)

    mesh = plsc.VectorSubcoreMesh(core_axis_name="c", subcore_axis_name="s",
                                  num_cores=NC, num_subcores=NS)

    deg_fn = functools.partial(
        pl.kernel,
        out_type=jax.ShapeDtypeStruct((NC * 2 * npad,), jnp.float32),
        mesh=mesh,
        scratch_types=[
            pltpu.VMEM((nch, chunk), jnp.int32),
            pltpu.VMEM((nch, chunk), jnp.int32),
            pltpu.VMEM((chunk,), jnp.float32),
            pltpu.VMEM((deg_pt,), jnp.float32),
            pltpu.VMEM((deg_pt,), jnp.float32),
            pltpu.VMEM_SHARED((npad,), jnp.float32),
            pltpu.VMEM_SHARED((npad,), jnp.float32),
        ],
    )(_deg_body)
    degp = deg_fn(srcr, dstr,
                  jnp.ones((chunk,), jnp.float32),
                  jnp.zeros((deg_pt,), jnp.float32)).reshape(NC, 2, npad)

    p0 = degp[0, 0, :n].reshape(n, 1)
    p1 = degp[1, 0, :n].reshape(n, 1)
    q0 = degp[0, 1, :n].reshape(n, 1)
    q1 = degp[1, 1, :n].reshape(n, 1)

    blk = 1000
    grid = (n // blk,)
    ys = pl.pallas_call(
        _mm_body,
        grid=grid,
        in_specs=[
            pl.BlockSpec((blk, d), lambda i: (i, 0)),
            pl.BlockSpec((d, d_out), lambda i: (0, 0)),
            pl.BlockSpec((blk, 1), lambda i: (i, 0)),
            pl.BlockSpec((blk, 1), lambda i: (i, 0)),
        ],
        out_specs=pl.BlockSpec((blk, d_out), lambda i: (i, 0)),
        out_shape=jax.ShapeDtypeStruct((n, d_out), jnp.float32),
    )(features, W, p0, p1)

    agg_fn = functools.partial(
        pl.kernel,
        out_type=jax.ShapeDtypeStruct((NC, npad, d_out), jnp.float32),
        mesh=mesh,
        scratch_types=[
            pltpu.VMEM((nch, chunk), jnp.int32),
            pltpu.VMEM((nch, chunk), jnp.int32),
            pltpu.VMEM((chunk, d_out), jnp.float32),
            pltpu.VMEM_SHARED((npad, d_out), jnp.float32),
            pltpu.SemaphoreType.DMA,
        ],
    )(_agg_body)
    parts = agg_fn(ys, srcr, dstr,
                   jnp.zeros((chunk, d_out), jnp.float32))

    out = pl.pallas_call(
        _fin_body,
        grid=grid,
        in_specs=[
            pl.BlockSpec((blk, d_out), lambda i: (i, 0)),
            pl.BlockSpec((blk, d_out), lambda i: (i, 0)),
            pl.BlockSpec((blk, 1), lambda i: (i, 0)),
            pl.BlockSpec((blk, 1), lambda i: (i, 0)),
            pl.BlockSpec((1, d_out), lambda i: (0, 0)),
        ],
        out_specs=pl.BlockSpec((blk, d_out), lambda i: (i, 0)),
        out_shape=jax.ShapeDtypeStruct((n, d_out), jnp.float32),
    )(parts[0, :n], parts[1, :n], q0, q1, b.reshape(1, d_out))

    return (g, out)


# speedup vs baseline: 10.6215x; 1.3572x over previous
import functools

import jax
import jax.numpy as jnp
from jax import lax
from jax.experimental import pallas as pl
from jax.experimental.pallas import tpu as pltpu
from jax.experimental.pallas import tpu_sc as plsc

NC = 2
NS = 16
NW = NC * NS


def _deg_body(srcr, dstr, ones_h, zeros_h, out, sidx, didx, ones_v, zbuf,
              dbuf, acc_s, acc_d):
    npad = acc_s.shape[0]
    per_tile = npad // NS
    cid = lax.axis_index("c")
    sid = lax.axis_index("s")
    wid = cid * NS + sid
    nch = sidx.shape[0]

    pltpu.sync_copy(srcr.at[wid], sidx)
    pltpu.sync_copy(dstr.at[wid], didx)
    pltpu.sync_copy(ones_h, ones_v)
    pltpu.sync_copy(zeros_h, zbuf)

    base = sid * per_tile
    pltpu.sync_copy(zbuf, acc_s.at[pl.ds(base, per_tile)])
    pltpu.sync_copy(zbuf, acc_d.at[pl.ds(base, per_tile)])
    plsc.subcore_barrier()

    def body(j, carry):
        pltpu.sync_copy(ones_v, acc_s.at[sidx.at[j]], add=True)
        pltpu.sync_copy(ones_v, acc_d.at[didx.at[j]], add=True)
        return carry

    lax.fori_loop(0, nch, body, 0)
    plsc.subcore_barrier()

    pltpu.sync_copy(acc_s.at[pl.ds(base, per_tile)], dbuf)
    pltpu.sync_copy(dbuf, out.at[pl.ds(cid * 2 * npad + base, per_tile)])
    pltpu.sync_copy(acc_d.at[pl.ds(base, per_tile)], dbuf)
    pltpu.sync_copy(dbuf, out.at[pl.ds((cid * 2 + 1) * npad + base,
                                       per_tile)])


def _agg_body(ys, srcr, dstr, zeros_h, out,
              sb0, sb1, db0, db1, r0, r1, acc,
              ss0, ss1, ds0, ds1, gs0, gs1):
    npad, dd = acc.shape
    rows_per_tile = npad // NS
    chunk = r0.shape[0]
    nch = srcr.shape[1]
    ncopy = rows_per_tile // chunk
    cid = lax.axis_index("c")
    sid = lax.axis_index("s")
    wid = cid * NS + sid

    sb = (sb0, sb1)
    db = (db0, db1)
    rows = (r0, r1)
    ssem = (ss0, ss1)
    dsem = (ds0, ds1)
    gsem = (gs0, gs1)

    pltpu.sync_copy(zeros_h, r0)
    rowbase = sid * rows_per_tile
    for k in range(ncopy):
        sl = pl.ds(rowbase + k * chunk, chunk)
        pltpu.async_copy(r0, acc.at[sl], gs0)
    for k in range(ncopy):
        sl = pl.ds(rowbase + k * chunk, chunk)
        pltpu.make_async_copy(r0, acc.at[sl], gs0).wait()
    plsc.subcore_barrier()

    for b in range(2):
        pltpu.async_copy(srcr.at[wid, b], sb[b], ssem[b])
        pltpu.async_copy(dstr.at[wid, b], db[b], dsem[b])
    for b in range(2):
        pltpu.make_async_copy(srcr.at[wid, b], sb[b], ssem[b]).wait()
        pltpu.async_copy(ys.at[sb[b]], rows[b], gsem[b])

    def group(gi, carry):
        for b in range(2):
            j = 2 * gi + b
            pltpu.make_async_copy(ys.at[pl.ds(0, chunk)], rows[b],
                                  gsem[b]).wait()

            @pl.when(j + 2 < nch)
            def _():
                pltpu.async_copy(srcr.at[wid, j + 2], sb[b], ssem[b])

            pltpu.make_async_copy(dstr.at[wid, 0], db[b], dsem[b]).wait()
            pltpu.sync_copy(rows[b], acc.at[db[b]], add=True)

            @pl.when(j + 2 < nch)
            def _():
                pltpu.async_copy(dstr.at[wid, j + 2], db[b], dsem[b])
                pltpu.make_async_copy(srcr.at[wid, 0], sb[b],
                                      ssem[b]).wait()
                pltpu.async_copy(ys.at[sb[b]], rows[b], gsem[b])

        return carry

    lax.fori_loop(0, nch // 2, group, 0)
    if nch % 2 == 1:
        b = (nch - 1) % 2
        pltpu.make_async_copy(ys.at[pl.ds(0, chunk)], rows[b],
                              gsem[b]).wait()
        pltpu.make_async_copy(dstr.at[wid, 0], db[b], dsem[b]).wait()
        pltpu.sync_copy(rows[b], acc.at[db[b]], add=True)
    plsc.subcore_barrier()

    for k in range(ncopy):
        b = k % 2
        sl = pl.ds(rowbase + k * chunk, chunk)
        if k >= 2:
            slp = pl.ds(rowbase + (k - 2) * chunk, chunk)
            pltpu.make_async_copy(rows[b], out.at[cid, slp],
                                  gsem[b]).wait()
        pltpu.sync_copy(acc.at[sl], rows[b])
        pltpu.async_copy(rows[b], out.at[cid, sl], gsem[b])
    for k in range(max(ncopy - 2, 0), ncopy):
        b = k % 2
        sl = pl.ds(rowbase + k * chunk, chunk)
        pltpu.make_async_copy(rows[b], out.at[cid, sl], gsem[b]).wait()


def _mm_body(x_ref, w_ref, p0_ref, p1_ref, o_ref):
    deg = p0_ref[0] + p1_ref[0]
    norm = lax.rsqrt(jnp.maximum(deg, 1.0))
    y = jnp.dot(x_ref[...], w_ref[...], preferred_element_type=jnp.float32)
    o_ref[...] = y * norm


def _fin_body(a0_ref, a1_ref, q0_ref, q1_ref, b_ref, o_ref):
    deg = q0_ref[0] + q1_ref[0]
    norm = lax.rsqrt(jnp.maximum(deg, 1.0))
    s = (a0_ref[0] + a1_ref[0]) * norm + b_ref[...]
    o_ref[...] = jnp.maximum(s, 0.0)


def kernel(g, features, W, b):
    n, d = features.shape
    e = g.shape[1]
    d_out = W.shape[1]

    chunk = 80
    assert e % (NW * chunk) == 0
    nch = e // (NW * chunk)
    npad = ((n + NS * 128 - 1) // (NS * 128)) * (NS * 128)
    deg_pt = npad // NS
    assert (npad // NS) % chunk == 0

    g32 = g.astype(jnp.int32)
    srcr = g32[0].reshape(NW, nch, chunk)
    dstr = g32[1].reshape(NW, nch, chunk)

    mesh = plsc.VectorSubcoreMesh(core_axis_name="c", subcore_axis_name="s",
                                  num_cores=NC, num_subcores=NS)

    deg_fn = functools.partial(
        pl.kernel,
        out_type=jax.ShapeDtypeStruct((NC * 2 * npad,), jnp.float32),
        mesh=mesh,
        scratch_types=[
            pltpu.VMEM((nch, chunk), jnp.int32),
            pltpu.VMEM((nch, chunk), jnp.int32),
            pltpu.VMEM((chunk,), jnp.float32),
            pltpu.VMEM((deg_pt,), jnp.float32),
            pltpu.VMEM((deg_pt,), jnp.float32),
            pltpu.VMEM_SHARED((npad,), jnp.float32),
            pltpu.VMEM_SHARED((npad,), jnp.float32),
        ],
    )(_deg_body)
    degp = deg_fn(srcr, dstr,
                  jnp.ones((chunk,), jnp.float32),
                  jnp.zeros((deg_pt,), jnp.float32))
    deg4 = degp.reshape(4, npad, 1)

    blk = 1000
    grid = (n // blk,)
    ys = pl.pallas_call(
        _mm_body,
        grid=grid,
        in_specs=[
            pl.BlockSpec((blk, d), lambda i: (i, 0)),
            pl.BlockSpec((d, d_out), lambda i: (0, 0)),
            pl.BlockSpec((1, blk, 1), lambda i: (0, i, 0)),
            pl.BlockSpec((1, blk, 1), lambda i: (2, i, 0)),
        ],
        out_specs=pl.BlockSpec((blk, d_out), lambda i: (i, 0)),
        out_shape=jax.ShapeDtypeStruct((n, d_out), jnp.float32),
    )(features, W, deg4, deg4)

    agg_fn = functools.partial(
        pl.kernel,
        out_type=jax.ShapeDtypeStruct((NC, npad, d_out), jnp.float32),
        mesh=mesh,
        scratch_types=[
            pltpu.VMEM((chunk,), jnp.int32),
            pltpu.VMEM((chunk,), jnp.int32),
            pltpu.VMEM((chunk,), jnp.int32),
            pltpu.VMEM((chunk,), jnp.int32),
            pltpu.VMEM((chunk, d_out), jnp.float32),
            pltpu.VMEM((chunk, d_out), jnp.float32),
            pltpu.VMEM_SHARED((npad, d_out), jnp.float32),
            pltpu.SemaphoreType.DMA,
            pltpu.SemaphoreType.DMA,
            pltpu.SemaphoreType.DMA,
            pltpu.SemaphoreType.DMA,
            pltpu.SemaphoreType.DMA,
            pltpu.SemaphoreType.DMA,
        ],
    )(_agg_body)
    parts = agg_fn(ys, srcr, dstr,
                   jnp.zeros((chunk, d_out), jnp.float32))

    out = pl.pallas_call(
        _fin_body,
        grid=grid,
        in_specs=[
            pl.BlockSpec((1, blk, d_out), lambda i: (0, i, 0)),
            pl.BlockSpec((1, blk, d_out), lambda i: (1, i, 0)),
            pl.BlockSpec((1, blk, 1), lambda i: (1, i, 0)),
            pl.BlockSpec((1, blk, 1), lambda i: (3, i, 0)),
            pl.BlockSpec((1, d_out), lambda i: (0, 0)),
        ],
        out_specs=pl.BlockSpec((blk, d_out), lambda i: (i, 0)),
        out_shape=jax.ShapeDtypeStruct((n, d_out), jnp.float32),
    )(parts, parts, deg4, deg4, b.reshape(1, d_out))

    return (g, out)


# speedup vs baseline: 12.4159x; 1.1689x over previous
import functools

import jax
import jax.numpy as jnp
from jax import lax
from jax.experimental import pallas as pl
from jax.experimental.pallas import tpu as pltpu
from jax.experimental.pallas import tpu_sc as plsc

NC = 2
NS = 16
NW = NC * NS


def _deg_body(srcr, dstr, ones_h, zeros_h, out, sidx, didx, ones_v, zbuf,
              dbuf, acc_s, acc_d, ss, sd):
    npad = acc_s.shape[0]
    per_tile = npad // NS
    cid = lax.axis_index("c")
    sid = lax.axis_index("s")
    wid = cid * NS + sid
    nch = sidx.shape[0]
    depth = 8

    pltpu.sync_copy(srcr.at[wid], sidx)
    pltpu.sync_copy(dstr.at[wid], didx)
    pltpu.sync_copy(ones_h, ones_v)
    pltpu.sync_copy(zeros_h, zbuf)

    base = sid * per_tile
    pltpu.sync_copy(zbuf, acc_s.at[pl.ds(base, per_tile)])
    pltpu.sync_copy(zbuf, acc_d.at[pl.ds(base, per_tile)])
    plsc.subcore_barrier()

    def body(j, carry):
        pltpu.async_copy(ones_v, acc_s.at[sidx.at[j]], ss, add=True)
        pltpu.async_copy(ones_v, acc_d.at[didx.at[j]], sd, add=True)

        @pl.when(j >= depth)
        def _():
            pltpu.make_async_copy(ones_v, acc_s.at[sidx.at[0]], ss).wait()
            pltpu.make_async_copy(ones_v, acc_d.at[didx.at[0]], sd).wait()

        return carry

    lax.fori_loop(0, nch, body, 0)

    def drain(j, carry):
        pltpu.make_async_copy(ones_v, acc_s.at[sidx.at[0]], ss).wait()
        pltpu.make_async_copy(ones_v, acc_d.at[didx.at[0]], sd).wait()
        return carry

    lax.fori_loop(0, min(depth, nch), drain, 0)
    plsc.subcore_barrier()

    pltpu.sync_copy(acc_s.at[pl.ds(base, per_tile)], dbuf)
    pltpu.sync_copy(dbuf, out.at[pl.ds(cid * 2 * npad + base, per_tile)])
    pltpu.sync_copy(acc_d.at[pl.ds(base, per_tile)], dbuf)
    pltpu.sync_copy(dbuf, out.at[pl.ds((cid * 2 + 1) * npad + base,
                                       per_tile)])


def _agg_body(ys, srcr, dstr, zeros_h, out,
              sb0, sb1, db0, db1, r0, r1, acc,
              ss0, ss1, ds0, ds1, gs0, gs1):
    npad, dd = acc.shape
    rows_per_tile = npad // NS
    chunk = r0.shape[0]
    nch = srcr.shape[1]
    ncopy = rows_per_tile // chunk
    cid = lax.axis_index("c")
    sid = lax.axis_index("s")
    wid = cid * NS + sid

    sb = (sb0, sb1)
    db = (db0, db1)
    rows = (r0, r1)
    ssem = (ss0, ss1)
    dsem = (ds0, ds1)
    gsem = (gs0, gs1)

    pltpu.sync_copy(zeros_h, r0)
    rowbase = sid * rows_per_tile
    for k in range(ncopy):
        sl = pl.ds(rowbase + k * chunk, chunk)
        pltpu.async_copy(r0, acc.at[sl], gs0)
    for k in range(ncopy):
        sl = pl.ds(rowbase + k * chunk, chunk)
        pltpu.make_async_copy(r0, acc.at[sl], gs0).wait()
    plsc.subcore_barrier()

    for b in range(2):
        pltpu.async_copy(srcr.at[wid, b], sb[b], ssem[b])
        pltpu.async_copy(dstr.at[wid, b], db[b], dsem[b])
    for b in range(2):
        pltpu.make_async_copy(srcr.at[wid, b], sb[b], ssem[b]).wait()
        pltpu.async_copy(ys.at[sb[b]], rows[b], gsem[b])

    def group(gi, carry):
        for b in range(2):
            j = 2 * gi + b
            pltpu.make_async_copy(ys.at[pl.ds(0, chunk)], rows[b],
                                  gsem[b]).wait()

            @pl.when(j + 2 < nch)
            def _():
                pltpu.async_copy(srcr.at[wid, j + 2], sb[b], ssem[b])

            pltpu.make_async_copy(dstr.at[wid, 0], db[b], dsem[b]).wait()
            pltpu.sync_copy(rows[b], acc.at[db[b]], add=True)

            @pl.when(j + 2 < nch)
            def _():
                pltpu.async_copy(dstr.at[wid, j + 2], db[b], dsem[b])
                pltpu.make_async_copy(srcr.at[wid, 0], sb[b],
                                      ssem[b]).wait()
                pltpu.async_copy(ys.at[sb[b]], rows[b], gsem[b])

        return carry

    lax.fori_loop(0, nch // 2, group, 0)
    if nch % 2 == 1:
        b = (nch - 1) % 2
        pltpu.make_async_copy(ys.at[pl.ds(0, chunk)], rows[b],
                              gsem[b]).wait()
        pltpu.make_async_copy(dstr.at[wid, 0], db[b], dsem[b]).wait()
        pltpu.sync_copy(rows[b], acc.at[db[b]], add=True)
    plsc.subcore_barrier()

    for k in range(ncopy):
        b = k % 2
        sl = pl.ds(rowbase + k * chunk, chunk)
        if k >= 2:
            slp = pl.ds(rowbase + (k - 2) * chunk, chunk)
            pltpu.make_async_copy(rows[b], out.at[cid, slp],
                                  gsem[b]).wait()
        pltpu.sync_copy(acc.at[sl], rows[b])
        pltpu.async_copy(rows[b], out.at[cid, sl], gsem[b])
    for k in range(max(ncopy - 2, 0), ncopy):
        b = k % 2
        sl = pl.ds(rowbase + k * chunk, chunk)
        pltpu.make_async_copy(rows[b], out.at[cid, sl], gsem[b]).wait()


def _mm_body(x_ref, w_ref, o_ref):
    o_ref[...] = jnp.dot(x_ref[...], w_ref[...],
                         preferred_element_type=jnp.float32)


def _scale_body(z_ref, deg_ref, o_ref):
    norm = lax.rsqrt(jnp.maximum(deg_ref[:, 0:1], 1.0))
    o_ref[...] = z_ref[...] * norm


def _fin_body(a0_ref, a1_ref, deg_ref, b_ref, o_ref):
    norm = lax.rsqrt(jnp.maximum(deg_ref[:, 1:2], 1.0))
    s = (a0_ref[0] + a1_ref[0]) * norm + b_ref[...]
    o_ref[...] = jnp.maximum(s, 0.0)


def kernel(g, features, W, b):
    n, d = features.shape
    e = g.shape[1]
    d_out = W.shape[1]

    chunk = 80
    assert e % (NW * chunk) == 0
    nch = e // (NW * chunk)
    npad = ((n + NS * 128 - 1) // (NS * 128)) * (NS * 128)
    deg_pt = npad // NS
    assert (npad // NS) % chunk == 0

    g32 = g.astype(jnp.int32)
    srcr = g32[0].reshape(NW, nch, chunk)
    dstr = g32[1].reshape(NW, nch, chunk)

    mesh = plsc.VectorSubcoreMesh(core_axis_name="c", subcore_axis_name="s",
                                  num_cores=NC, num_subcores=NS)

    deg_fn = functools.partial(
        pl.kernel,
        out_type=jax.ShapeDtypeStruct((NC * 2 * npad,), jnp.float32),
        mesh=mesh,
        scratch_types=[
            pltpu.VMEM((nch, chunk), jnp.int32),
            pltpu.VMEM((nch, chunk), jnp.int32),
            pltpu.VMEM((chunk,), jnp.float32),
            pltpu.VMEM((deg_pt,), jnp.float32),
            pltpu.VMEM((deg_pt,), jnp.float32),
            pltpu.VMEM_SHARED((npad,), jnp.float32),
            pltpu.VMEM_SHARED((npad,), jnp.float32),
            pltpu.SemaphoreType.DMA,
            pltpu.SemaphoreType.DMA,
        ],
    )(_deg_body)
    degp = deg_fn(srcr, dstr,
                  jnp.ones((chunk,), jnp.float32),
                  jnp.zeros((deg_pt,), jnp.float32))
    deg2t = degp.reshape(2, 2, npad).sum(axis=0).T

    blk = 1000
    grid = (n // blk,)
    z = pl.pallas_call(
        _mm_body,
        grid=grid,
        in_specs=[
            pl.BlockSpec((blk, d), lambda i: (i, 0)),
            pl.BlockSpec((d, d_out), lambda i: (0, 0)),
        ],
        out_specs=pl.BlockSpec((blk, d_out), lambda i: (i, 0)),
        out_shape=jax.ShapeDtypeStruct((n, d_out), jnp.float32),
    )(features, W)

    ys = pl.pallas_call(
        _scale_body,
        grid=grid,
        in_specs=[
            pl.BlockSpec((blk, d_out), lambda i: (i, 0)),
            pl.BlockSpec((blk, 2), lambda i: (i, 0)),
        ],
        out_specs=pl.BlockSpec((blk, d_out), lambda i: (i, 0)),
        out_shape=jax.ShapeDtypeStruct((n, d_out), jnp.float32),
    )(z, deg2t)

    agg_fn = functools.partial(
        pl.kernel,
        out_type=jax.ShapeDtypeStruct((NC, npad, d_out), jnp.float32),
        mesh=mesh,
        scratch_types=[
            pltpu.VMEM((chunk,), jnp.int32),
            pltpu.VMEM((chunk,), jnp.int32),
            pltpu.VMEM((chunk,), jnp.int32),
            pltpu.VMEM((chunk,), jnp.int32),
            pltpu.VMEM((chunk, d_out), jnp.float32),
            pltpu.VMEM((chunk, d_out), jnp.float32),
            pltpu.VMEM_SHARED((npad, d_out), jnp.float32),
            pltpu.SemaphoreType.DMA,
            pltpu.SemaphoreType.DMA,
            pltpu.SemaphoreType.DMA,
            pltpu.SemaphoreType.DMA,
            pltpu.SemaphoreType.DMA,
            pltpu.SemaphoreType.DMA,
        ],
    )(_agg_body)
    parts = agg_fn(ys, srcr, dstr,
                   jnp.zeros((chunk, d_out), jnp.float32))

    out = pl.pallas_call(
        _fin_body,
        grid=grid,
        in_specs=[
            pl.BlockSpec((1, blk, d_out), lambda i: (0, i, 0)),
            pl.BlockSpec((1, blk, d_out), lambda i: (1, i, 0)),
            pl.BlockSpec((blk, 2), lambda i: (i, 0)),
            pl.BlockSpec((1, d_out), lambda i: (0, 0)),
        ],
        out_specs=pl.BlockSpec((blk, d_out), lambda i: (i, 0)),
        out_shape=jax.ShapeDtypeStruct((n, d_out), jnp.float32),
    )(parts, parts, deg2t, b.reshape(1, d_out))

    return (g, out)


# speedup vs baseline: 14.0260x; 1.1297x over previous
import functools

import jax
import jax.numpy as jnp
from jax import lax
from jax.experimental import pallas as pl
from jax.experimental.pallas import tpu as pltpu
from jax.experimental.pallas import tpu_sc as plsc

NC = 2
NS = 16
NW = NC * NS


def _deg_body(srcr, dstr, ones_h, zeros_h, out, sidx, didx, ones_v, zbuf,
              dbuf, acc_s, acc_d, ss, sd):
    npad = acc_s.shape[0]
    per_tile = npad // NS
    cid = lax.axis_index("c")
    sid = lax.axis_index("s")
    wid = cid * NS + sid
    nch = sidx.shape[0]
    depth = 8

    pltpu.sync_copy(srcr.at[wid], sidx)
    pltpu.sync_copy(dstr.at[wid], didx)
    pltpu.sync_copy(ones_h, ones_v)
    pltpu.sync_copy(zeros_h, zbuf)

    base = sid * per_tile
    pltpu.sync_copy(zbuf, acc_s.at[pl.ds(base, per_tile)])
    pltpu.sync_copy(zbuf, acc_d.at[pl.ds(base, per_tile)])
    plsc.subcore_barrier()

    def body(j, carry):
        pltpu.async_copy(ones_v, acc_s.at[sidx.at[j]], ss, add=True)
        pltpu.async_copy(ones_v, acc_d.at[didx.at[j]], sd, add=True)

        @pl.when(j >= depth)
        def _():
            pltpu.make_async_copy(ones_v, acc_s.at[sidx.at[0]], ss).wait()
            pltpu.make_async_copy(ones_v, acc_d.at[didx.at[0]], sd).wait()

        return carry

    lax.fori_loop(0, nch, body, 0)

    def drain(j, carry):
        pltpu.make_async_copy(ones_v, acc_s.at[sidx.at[0]], ss).wait()
        pltpu.make_async_copy(ones_v, acc_d.at[didx.at[0]], sd).wait()
        return carry

    lax.fori_loop(0, min(depth, nch), drain, 0)
    plsc.subcore_barrier()

    pltpu.sync_copy(acc_s.at[pl.ds(base, per_tile)], dbuf)
    pltpu.sync_copy(dbuf, out.at[pl.ds(cid * 2 * npad + base, per_tile)])
    pltpu.sync_copy(acc_d.at[pl.ds(base, per_tile)], dbuf)
    pltpu.sync_copy(dbuf, out.at[pl.ds((cid * 2 + 1) * npad + base,
                                       per_tile)])


NBUF = 4


def _agg_body(ys, srcr, dstr, zeros_h, out,
              sb0, sb1, sb2, sb3, db0, db1, db2, db3, r0, r1, r2, r3, acc,
              ss0, ss1, ss2, ss3, ds0, ds1, ds2, ds3,
              gs0, gs1, gs2, gs3, cs0, cs1, cs2, cs3):
    npad, dd = acc.shape
    rows_per_tile = npad // NS
    chunk = r0.shape[0]
    nch = srcr.shape[1]
    ncopy = rows_per_tile // chunk
    cid = lax.axis_index("c")
    sid = lax.axis_index("s")
    wid = cid * NS + sid

    sb = (sb0, sb1, sb2, sb3)
    db = (db0, db1, db2, db3)
    rows = (r0, r1, r2, r3)
    ssem = (ss0, ss1, ss2, ss3)
    dsem = (ds0, ds1, ds2, ds3)
    gsem = (gs0, gs1, gs2, gs3)
    csem = (cs0, cs1, cs2, cs3)

    pltpu.sync_copy(zeros_h, r0)
    rowbase = sid * rows_per_tile
    for k in range(ncopy):
        sl = pl.ds(rowbase + k * chunk, chunk)
        pltpu.async_copy(r0, acc.at[sl], gs0)
    for k in range(ncopy):
        sl = pl.ds(rowbase + k * chunk, chunk)
        pltpu.make_async_copy(r0, acc.at[sl], gs0).wait()
    plsc.subcore_barrier()

    def _prep(t, j):
        pltpu.async_copy(srcr.at[wid, j], sb[t], ssem[t])
        pltpu.async_copy(dstr.at[wid, j], db[t], dsem[t])
        pltpu.make_async_copy(srcr.at[wid, 0], sb[t], ssem[t]).wait()
        pltpu.async_copy(ys.at[sb[t]], rows[t], gsem[t])

    for b in range(min(3, nch)):
        _prep(b, b)

    def group(gi, carry):
        for b in range(NBUF):
            j = NBUF * gi + b
            s = b
            pltpu.make_async_copy(ys.at[pl.ds(0, chunk)], rows[s],
                                  gsem[s]).wait()
            pltpu.make_async_copy(dstr.at[wid, 0], db[s], dsem[s]).wait()
            pltpu.async_copy(rows[s], acc.at[db[s]], csem[s], add=True)
            t = (b + 3) % NBUF

            @pl.when(j + 3 < nch)
            def _():
                @pl.when(j >= 1)
                def _():
                    pltpu.make_async_copy(rows[t], acc.at[db[t]],
                                          csem[t]).wait()

                _prep(t, j + 3)

        return carry

    ngroups = (nch - 1) // NBUF
    lax.fori_loop(0, ngroups, group, 0)
    for j in range(ngroups * NBUF, nch):
        s = j % NBUF
        pltpu.make_async_copy(ys.at[pl.ds(0, chunk)], rows[s],
                              gsem[s]).wait()
        pltpu.make_async_copy(dstr.at[wid, 0], db[s], dsem[s]).wait()
        pltpu.async_copy(rows[s], acc.at[db[s]], csem[s], add=True)
    for j in range(max(nch - NBUF, 0), nch):
        s = j % NBUF
        pltpu.make_async_copy(rows[s], acc.at[db[s]], csem[s]).wait()
    plsc.subcore_barrier()

    for k in range(ncopy):
        b = k % 2
        sl = pl.ds(rowbase + k * chunk, chunk)
        if k >= 2:
            slp = pl.ds(rowbase + (k - 2) * chunk, chunk)
            pltpu.make_async_copy(rows[b], out.at[cid, slp],
                                  gsem[b]).wait()
        pltpu.sync_copy(acc.at[sl], rows[b])
        pltpu.async_copy(rows[b], out.at[cid, sl], gsem[b])
    for k in range(max(ncopy - 2, 0), ncopy):
        b = k % 2
        sl = pl.ds(rowbase + k * chunk, chunk)
        pltpu.make_async_copy(rows[b], out.at[cid, sl], gsem[b]).wait()


def _mm_body(x_ref, w_ref, o_ref):
    o_ref[...] = jnp.dot(x_ref[...], w_ref[...],
                         preferred_element_type=jnp.float32)


def _scale_body(z_ref, deg_ref, o_ref):
    norm = lax.rsqrt(jnp.maximum(deg_ref[:, 0:1], 1.0))
    o_ref[...] = z_ref[...] * norm


def _fin_body(a0_ref, a1_ref, deg_ref, b_ref, o_ref):
    norm = lax.rsqrt(jnp.maximum(deg_ref[:, 1:2], 1.0))
    s = (a0_ref[0] + a1_ref[0]) * norm + b_ref[...]
    o_ref[...] = jnp.maximum(s, 0.0)


def kernel(g, features, W, b):
    n, d = features.shape
    e = g.shape[1]
    d_out = W.shape[1]

    chunk = 80
    assert e % (NW * chunk) == 0
    nch = e // (NW * chunk)
    npad = ((n + NS * 128 - 1) // (NS * 128)) * (NS * 128)
    deg_pt = npad // NS
    assert (npad // NS) % chunk == 0

    g32 = g.astype(jnp.int32)
    srcr = g32[0].reshape(NW, nch, chunk)
    dstr = g32[1].reshape(NW, nch, chunk)

    mesh = plsc.VectorSubcoreMesh(core_axis_name="c", subcore_axis_name="s",
                                  num_cores=NC, num_subcores=NS)

    deg_fn = functools.partial(
        pl.kernel,
        out_type=jax.ShapeDtypeStruct((NC * 2 * npad,), jnp.float32),
        mesh=mesh,
        scratch_types=[
            pltpu.VMEM((nch, chunk), jnp.int32),
            pltpu.VMEM((nch, chunk), jnp.int32),
            pltpu.VMEM((chunk,), jnp.float32),
            pltpu.VMEM((deg_pt,), jnp.float32),
            pltpu.VMEM((deg_pt,), jnp.float32),
            pltpu.VMEM_SHARED((npad,), jnp.float32),
            pltpu.VMEM_SHARED((npad,), jnp.float32),
            pltpu.SemaphoreType.DMA,
            pltpu.SemaphoreType.DMA,
        ],
    )(_deg_body)
    degp = deg_fn(srcr, dstr,
                  jnp.ones((chunk,), jnp.float32),
                  jnp.zeros((deg_pt,), jnp.float32))
    deg2t = degp.reshape(2, 2, npad).sum(axis=0).T

    blk = 1000
    grid = (n // blk,)
    z = pl.pallas_call(
        _mm_body,
        grid=grid,
        in_specs=[
            pl.BlockSpec((blk, d), lambda i: (i, 0)),
            pl.BlockSpec((d, d_out), lambda i: (0, 0)),
        ],
        out_specs=pl.BlockSpec((blk, d_out), lambda i: (i, 0)),
        out_shape=jax.ShapeDtypeStruct((n, d_out), jnp.float32),
    )(features, W)

    ys = pl.pallas_call(
        _scale_body,
        grid=grid,
        in_specs=[
            pl.BlockSpec((blk, d_out), lambda i: (i, 0)),
            pl.BlockSpec((blk, 2), lambda i: (i, 0)),
        ],
        out_specs=pl.BlockSpec((blk, d_out), lambda i: (i, 0)),
        out_shape=jax.ShapeDtypeStruct((n, d_out), jnp.float32),
    )(z, deg2t)

    agg_fn = functools.partial(
        pl.kernel,
        out_type=jax.ShapeDtypeStruct((NC, npad, d_out), jnp.float32),
        mesh=mesh,
        scratch_types=(
            [pltpu.VMEM((chunk,), jnp.int32) for _ in range(2 * NBUF)]
            + [pltpu.VMEM((chunk, d_out), jnp.float32) for _ in range(NBUF)]
            + [pltpu.VMEM_SHARED((npad, d_out), jnp.float32)]
            + [pltpu.SemaphoreType.DMA for _ in range(4 * NBUF)]
        ),
    )(_agg_body)
    parts = agg_fn(ys, srcr, dstr,
                   jnp.zeros((chunk, d_out), jnp.float32))

    out = pl.pallas_call(
        _fin_body,
        grid=grid,
        in_specs=[
            pl.BlockSpec((1, blk, d_out), lambda i: (0, i, 0)),
            pl.BlockSpec((1, blk, d_out), lambda i: (1, i, 0)),
            pl.BlockSpec((blk, 2), lambda i: (i, 0)),
            pl.BlockSpec((1, d_out), lambda i: (0, 0)),
        ],
        out_specs=pl.BlockSpec((blk, d_out), lambda i: (i, 0)),
        out_shape=jax.ShapeDtypeStruct((n, d_out), jnp.float32),
    )(parts, parts, deg2t, b.reshape(1, d_out))

    return (g, out)


# speedup vs baseline: 15.3543x; 1.0947x over previous
import functools

import jax
import jax.numpy as jnp
from jax import lax
from jax.experimental import pallas as pl
from jax.experimental.pallas import tpu as pltpu
from jax.experimental.pallas import tpu_sc as plsc

NC = 2
NS = 16
NW = NC * NS


def _deg_body(g4, ones_h, zeros_h, out, sidx, didx, ones_v, zbuf,
              dbuf, acc_s, acc_d, ss, sd):
    npad = acc_s.shape[0]
    per_tile = npad // NS
    cid = lax.axis_index("c")
    sid = lax.axis_index("s")
    wid = cid * NS + sid
    nch = sidx.shape[0]
    depth = 8

    pltpu.sync_copy(g4.at[0, wid], sidx)
    pltpu.sync_copy(g4.at[1, wid], didx)
    pltpu.sync_copy(ones_h, ones_v)
    pltpu.sync_copy(zeros_h, zbuf)

    base = sid * per_tile
    pltpu.sync_copy(zbuf, acc_s.at[pl.ds(base, per_tile)])
    pltpu.sync_copy(zbuf, acc_d.at[pl.ds(base, per_tile)])
    plsc.subcore_barrier()

    def body(j, carry):
        pltpu.async_copy(ones_v, acc_s.at[sidx.at[j]], ss, add=True)
        pltpu.async_copy(ones_v, acc_d.at[didx.at[j]], sd, add=True)

        @pl.when(j >= depth)
        def _():
            pltpu.make_async_copy(ones_v, acc_s.at[sidx.at[0]], ss).wait()
            pltpu.make_async_copy(ones_v, acc_d.at[didx.at[0]], sd).wait()

        return carry

    lax.fori_loop(0, nch, body, 0)

    def drain(j, carry):
        pltpu.make_async_copy(ones_v, acc_s.at[sidx.at[0]], ss).wait()
        pltpu.make_async_copy(ones_v, acc_d.at[didx.at[0]], sd).wait()
        return carry

    lax.fori_loop(0, min(depth, nch), drain, 0)
    plsc.subcore_barrier()

    pltpu.sync_copy(acc_s.at[pl.ds(base, per_tile)], dbuf)
    pltpu.sync_copy(dbuf, out.at[pl.ds(cid * 2 * npad + base, per_tile)])
    pltpu.sync_copy(acc_d.at[pl.ds(base, per_tile)], dbuf)
    pltpu.sync_copy(dbuf, out.at[pl.ds((cid * 2 + 1) * npad + base,
                                       per_tile)])


NBUF = 4


def _agg_body(ys, g4, zeros_h, out,
              sb0, sb1, sb2, sb3, db0, db1, db2, db3, r0, r1, r2, r3, acc,
              ss0, ss1, ss2, ss3, ds0, ds1, ds2, ds3,
              gs0, gs1, gs2, gs3, cs0, cs1, cs2, cs3):
    npad, dd = acc.shape
    rows_per_tile = npad // NS
    chunk = r0.shape[0]
    nch = g4.shape[2]
    ncopy = rows_per_tile // chunk
    cid = lax.axis_index("c")
    sid = lax.axis_index("s")
    wid = cid * NS + sid

    sb = (sb0, sb1, sb2, sb3)
    db = (db0, db1, db2, db3)
    rows = (r0, r1, r2, r3)
    ssem = (ss0, ss1, ss2, ss3)
    dsem = (ds0, ds1, ds2, ds3)
    gsem = (gs0, gs1, gs2, gs3)
    csem = (cs0, cs1, cs2, cs3)

    pltpu.sync_copy(zeros_h, r0)
    rowbase = sid * rows_per_tile
    for k in range(ncopy):
        sl = pl.ds(rowbase + k * chunk, chunk)
        pltpu.async_copy(r0, acc.at[sl], gs0)
    for k in range(ncopy):
        sl = pl.ds(rowbase + k * chunk, chunk)
        pltpu.make_async_copy(r0, acc.at[sl], gs0).wait()
    plsc.subcore_barrier()

    def _prep(t, j):
        pltpu.async_copy(g4.at[0, wid, j], sb[t], ssem[t])
        pltpu.async_copy(g4.at[1, wid, j], db[t], dsem[t])
        pltpu.make_async_copy(g4.at[0, wid, 0], sb[t], ssem[t]).wait()
        pltpu.async_copy(ys.at[sb[t]], rows[t], gsem[t])

    for b in range(min(3, nch)):
        _prep(b, b)

    def group(gi, carry):
        for b in range(NBUF):
            j = NBUF * gi + b
            s = b
            pltpu.make_async_copy(ys.at[pl.ds(0, chunk)], rows[s],
                                  gsem[s]).wait()
            pltpu.make_async_copy(g4.at[1, wid, 0], db[s], dsem[s]).wait()
            pltpu.async_copy(rows[s], acc.at[db[s]], csem[s], add=True)
            t = (b + 3) % NBUF

            @pl.when(j + 3 < nch)
            def _():
                @pl.when(j >= 1)
                def _():
                    pltpu.make_async_copy(rows[t], acc.at[db[t]],
                                          csem[t]).wait()

                _prep(t, j + 3)

        return carry

    ngroups = (nch - 1) // NBUF
    lax.fori_loop(0, ngroups, group, 0)
    for j in range(ngroups * NBUF, nch):
        s = j % NBUF
        pltpu.make_async_copy(ys.at[pl.ds(0, chunk)], rows[s],
                              gsem[s]).wait()
        pltpu.make_async_copy(g4.at[1, wid, 0], db[s], dsem[s]).wait()
        pltpu.async_copy(rows[s], acc.at[db[s]], csem[s], add=True)
    for j in range(max(nch - NBUF, 0), nch):
        s = j % NBUF
        pltpu.make_async_copy(rows[s], acc.at[db[s]], csem[s]).wait()
    plsc.subcore_barrier()

    for k in range(ncopy):
        b = k % 2
        sl = pl.ds(rowbase + k * chunk, chunk)
        if k >= 2:
            slp = pl.ds(rowbase + (k - 2) * chunk, chunk)
            pltpu.make_async_copy(rows[b], out.at[cid, slp],
                                  gsem[b]).wait()
        pltpu.sync_copy(acc.at[sl], rows[b])
        pltpu.async_copy(rows[b], out.at[cid, sl], gsem[b])
    for k in range(max(ncopy - 2, 0), ncopy):
        b = k % 2
        sl = pl.ds(rowbase + k * chunk, chunk)
        pltpu.make_async_copy(rows[b], out.at[cid, sl], gsem[b]).wait()


def _mm_body(x_ref, w_ref, o_ref):
    o_ref[...] = jnp.dot(x_ref[...], w_ref[...],
                         preferred_element_type=jnp.float32)


def _scale_body(z_ref, deg_ref, o_ref):
    norm = lax.rsqrt(jnp.maximum(deg_ref[:, 0:1], 1.0))
    o_ref[...] = z_ref[...] * norm


def _fin_body(a0_ref, a1_ref, deg_ref, b_ref, o_ref):
    norm = lax.rsqrt(jnp.maximum(deg_ref[:, 1:2], 1.0))
    s = (a0_ref[0] + a1_ref[0]) * norm + b_ref[...]
    o_ref[...] = jnp.maximum(s, 0.0)


def kernel(g, features, W, b):
    n, d = features.shape
    e = g.shape[1]
    d_out = W.shape[1]

    chunk = 80
    assert e % (NW * chunk) == 0
    nch = e // (NW * chunk)
    npad = ((n + NS * 128 - 1) // (NS * 128)) * (NS * 128)
    deg_pt = npad // NS
    assert (npad // NS) % chunk == 0

    g32 = g.astype(jnp.int32)
    g4 = g32.reshape(2, NW, nch, chunk)

    mesh = plsc.VectorSubcoreMesh(core_axis_name="c", subcore_axis_name="s",
                                  num_cores=NC, num_subcores=NS)

    deg_fn = functools.partial(
        pl.kernel,
        out_type=jax.ShapeDtypeStruct((NC * 2 * npad,), jnp.float32),
        mesh=mesh,
        scratch_types=[
            pltpu.VMEM((nch, chunk), jnp.int32),
            pltpu.VMEM((nch, chunk), jnp.int32),
            pltpu.VMEM((chunk,), jnp.float32),
            pltpu.VMEM((deg_pt,), jnp.float32),
            pltpu.VMEM((deg_pt,), jnp.float32),
            pltpu.VMEM_SHARED((npad,), jnp.float32),
            pltpu.VMEM_SHARED((npad,), jnp.float32),
            pltpu.SemaphoreType.DMA,
            pltpu.SemaphoreType.DMA,
        ],
    )(_deg_body)
    degp = deg_fn(g4,
                  jnp.ones((chunk,), jnp.float32),
                  jnp.zeros((deg_pt,), jnp.float32))
    deg2t = degp.reshape(2, 2, npad).sum(axis=0).T

    blk = 2000
    grid = (n // blk,)
    z = pl.pallas_call(
        _mm_body,
        grid=grid,
        in_specs=[
            pl.BlockSpec((blk, d), lambda i: (i, 0)),
            pl.BlockSpec((d, d_out), lambda i: (0, 0)),
        ],
        out_specs=pl.BlockSpec((blk, d_out), lambda i: (i, 0)),
        out_shape=jax.ShapeDtypeStruct((n, d_out), jnp.float32),
    )(features, W)

    ys = pl.pallas_call(
        _scale_body,
        grid=grid,
        in_specs=[
            pl.BlockSpec((blk, d_out), lambda i: (i, 0)),
            pl.BlockSpec((blk, 2), lambda i: (i, 0)),
        ],
        out_specs=pl.BlockSpec((blk, d_out), lambda i: (i, 0)),
        out_shape=jax.ShapeDtypeStruct((n, d_out), jnp.float32),
    )(z, deg2t)

    agg_fn = functools.partial(
        pl.kernel,
        out_type=jax.ShapeDtypeStruct((NC, npad, d_out), jnp.float32),
        mesh=mesh,
        scratch_types=(
            [pltpu.VMEM((chunk,), jnp.int32) for _ in range(2 * NBUF)]
            + [pltpu.VMEM((chunk, d_out), jnp.float32) for _ in range(NBUF)]
            + [pltpu.VMEM_SHARED((npad, d_out), jnp.float32)]
            + [pltpu.SemaphoreType.DMA for _ in range(4 * NBUF)]
        ),
    )(_agg_body)
    parts = agg_fn(ys, g4,
                   jnp.zeros((chunk, d_out), jnp.float32))

    out = pl.pallas_call(
        _fin_body,
        grid=grid,
        in_specs=[
            pl.BlockSpec((1, blk, d_out), lambda i: (0, i, 0)),
            pl.BlockSpec((1, blk, d_out), lambda i: (1, i, 0)),
            pl.BlockSpec((blk, 2), lambda i: (i, 0)),
            pl.BlockSpec((1, d_out), lambda i: (0, 0)),
        ],
        out_specs=pl.BlockSpec((blk, d_out), lambda i: (i, 0)),
        out_shape=jax.ShapeDtypeStruct((n, d_out), jnp.float32),
    )(parts, parts, deg2t, b.reshape(1, d_out))

    return (g, out)


# speedup vs baseline: 15.3756x; 1.0014x over previous
import functools

import jax
import jax.numpy as jnp
from jax import lax
from jax.experimental import pallas as pl
from jax.experimental.pallas import tpu as pltpu
from jax.experimental.pallas import tpu_sc as plsc

NC = 2
NS = 16
NW = NC * NS


def _deg_body(g4, ones_h, zeros_h, out, sidx, didx, ones_v, zbuf,
              dbuf, acc_s, acc_d, ss, sd):
    npad = acc_s.shape[0]
    per_tile = npad // NS
    cid = lax.axis_index("c")
    sid = lax.axis_index("s")
    wid = cid * NS + sid
    nch = sidx.shape[0]
    depth = 8

    pltpu.sync_copy(g4.at[0, wid], sidx)
    pltpu.sync_copy(g4.at[1, wid], didx)
    pltpu.sync_copy(ones_h, ones_v)
    pltpu.sync_copy(zeros_h, zbuf)

    base = sid * per_tile
    pltpu.sync_copy(zbuf, acc_s.at[pl.ds(base, per_tile)])
    pltpu.sync_copy(zbuf, acc_d.at[pl.ds(base, per_tile)])
    plsc.subcore_barrier()

    unroll = 5
    ngr = nch // unroll
    assert nch % unroll == 0

    def body(gi, carry):
        for u in range(unroll):
            j = gi * unroll + u
            pltpu.async_copy(ones_v, acc_s.at[sidx.at[j]], ss, add=True)
            pltpu.async_copy(ones_v, acc_d.at[didx.at[j]], sd, add=True)

        @pl.when(gi >= depth)
        def _():
            for _u in range(unroll):
                pltpu.make_async_copy(ones_v, acc_s.at[sidx.at[0]],
                                      ss).wait()
                pltpu.make_async_copy(ones_v, acc_d.at[didx.at[0]],
                                      sd).wait()

        return carry

    lax.fori_loop(0, ngr, body, 0)

    def drain(gi, carry):
        for _u in range(unroll):
            pltpu.make_async_copy(ones_v, acc_s.at[sidx.at[0]], ss).wait()
            pltpu.make_async_copy(ones_v, acc_d.at[didx.at[0]], sd).wait()
        return carry

    lax.fori_loop(0, min(depth, ngr), drain, 0)
    plsc.subcore_barrier()

    pltpu.sync_copy(acc_s.at[pl.ds(base, per_tile)], dbuf)
    pltpu.sync_copy(dbuf, out.at[pl.ds(cid * 2 * npad + base, per_tile)])
    pltpu.sync_copy(acc_d.at[pl.ds(base, per_tile)], dbuf)
    pltpu.sync_copy(dbuf, out.at[pl.ds((cid * 2 + 1) * npad + base,
                                       per_tile)])


NBUF = 4


def _agg_body(ys, g4, zeros_h, out,
              sb0, sb1, sb2, sb3, db0, db1, db2, db3, r0, r1, r2, r3, acc,
              ss0, ss1, ss2, ss3, ds0, ds1, ds2, ds3,
              gs0, gs1, gs2, gs3, cs0, cs1, cs2, cs3):
    npad, dd = acc.shape
    rows_per_tile = npad // NS
    chunk = r0.shape[0]
    nch = g4.shape[2]
    ncopy = rows_per_tile // chunk
    cid = lax.axis_index("c")
    sid = lax.axis_index("s")
    wid = cid * NS + sid

    sb = (sb0, sb1, sb2, sb3)
    db = (db0, db1, db2, db3)
    rows = (r0, r1, r2, r3)
    ssem = (ss0, ss1, ss2, ss3)
    dsem = (ds0, ds1, ds2, ds3)
    gsem = (gs0, gs1, gs2, gs3)
    csem = (cs0, cs1, cs2, cs3)

    pltpu.sync_copy(zeros_h, r0)
    rowbase = sid * rows_per_tile
    for k in range(ncopy):
        sl = pl.ds(rowbase + k * chunk, chunk)
        pltpu.async_copy(r0, acc.at[sl], gs0)
    for k in range(ncopy):
        sl = pl.ds(rowbase + k * chunk, chunk)
        pltpu.make_async_copy(r0, acc.at[sl], gs0).wait()
    plsc.subcore_barrier()

    def _prep(t, j):
        pltpu.async_copy(g4.at[0, wid, j], sb[t], ssem[t])
        pltpu.async_copy(g4.at[1, wid, j], db[t], dsem[t])
        pltpu.make_async_copy(g4.at[0, wid, 0], sb[t], ssem[t]).wait()
        pltpu.async_copy(ys.at[sb[t]], rows[t], gsem[t])

    for b in range(min(3, nch)):
        _prep(b, b)

    def group(gi, carry):
        for b in range(NBUF):
            j = NBUF * gi + b
            s = b
            pltpu.make_async_copy(ys.at[pl.ds(0, chunk)], rows[s],
                                  gsem[s]).wait()
            pltpu.make_async_copy(g4.at[1, wid, 0], db[s], dsem[s]).wait()
            pltpu.async_copy(rows[s], acc.at[db[s]], csem[s], add=True)
            t = (b + 3) % NBUF

            @pl.when(j + 3 < nch)
            def _():
                @pl.when(j >= 1)
                def _():
                    pltpu.make_async_copy(rows[t], acc.at[db[t]],
                                          csem[t]).wait()

                _prep(t, j + 3)

        return carry

    ngroups = (nch - 1) // NBUF
    lax.fori_loop(0, ngroups, group, 0)
    for j in range(ngroups * NBUF, nch):
        s = j % NBUF
        pltpu.make_async_copy(ys.at[pl.ds(0, chunk)], rows[s],
                              gsem[s]).wait()
        pltpu.make_async_copy(g4.at[1, wid, 0], db[s], dsem[s]).wait()
        pltpu.async_copy(rows[s], acc.at[db[s]], csem[s], add=True)
    for j in range(max(nch - NBUF, 0), nch):
        s = j % NBUF
        pltpu.make_async_copy(rows[s], acc.at[db[s]], csem[s]).wait()
    plsc.subcore_barrier()

    for k in range(ncopy):
        b = k % 2
        sl = pl.ds(rowbase + k * chunk, chunk)
        if k >= 2:
            slp = pl.ds(rowbase + (k - 2) * chunk, chunk)
            pltpu.make_async_copy(rows[b], out.at[cid, slp],
                                  gsem[b]).wait()
        pltpu.sync_copy(acc.at[sl], rows[b])
        pltpu.async_copy(rows[b], out.at[cid, sl], gsem[b])
    for k in range(max(ncopy - 2, 0), ncopy):
        b = k % 2
        sl = pl.ds(rowbase + k * chunk, chunk)
        pltpu.make_async_copy(rows[b], out.at[cid, sl], gsem[b]).wait()


def _mm_body(x_ref, w_ref, o_ref):
    o_ref[...] = jnp.dot(x_ref[...], w_ref[...],
                         preferred_element_type=jnp.float32)


def _scale_body(z_ref, deg_ref, o_ref):
    norm = lax.rsqrt(jnp.maximum(deg_ref[:, 0:1], 1.0))
    o_ref[...] = z_ref[...] * norm


def _fin_body(a0_ref, a1_ref, deg_ref, b_ref, o_ref):
    norm = lax.rsqrt(jnp.maximum(deg_ref[:, 1:2], 1.0))
    s = (a0_ref[0] + a1_ref[0]) * norm + b_ref[...]
    o_ref[...] = jnp.maximum(s, 0.0)


def kernel(g, features, W, b):
    n, d = features.shape
    e = g.shape[1]
    d_out = W.shape[1]

    chunk = 80
    assert e % (NW * chunk) == 0
    nch = e // (NW * chunk)
    npad = ((n + NS * 128 - 1) // (NS * 128)) * (NS * 128)
    deg_pt = npad // NS
    assert (npad // NS) % chunk == 0

    g32 = g.astype(jnp.int32)
    g4 = g32.reshape(2, NW, nch, chunk)

    mesh = plsc.VectorSubcoreMesh(core_axis_name="c", subcore_axis_name="s",
                                  num_cores=NC, num_subcores=NS)

    deg_fn = functools.partial(
        pl.kernel,
        out_type=jax.ShapeDtypeStruct((NC * 2 * npad,), jnp.float32),
        mesh=mesh,
        scratch_types=[
            pltpu.VMEM((nch, chunk), jnp.int32),
            pltpu.VMEM((nch, chunk), jnp.int32),
            pltpu.VMEM((chunk,), jnp.float32),
            pltpu.VMEM((deg_pt,), jnp.float32),
            pltpu.VMEM((deg_pt,), jnp.float32),
            pltpu.VMEM_SHARED((npad,), jnp.float32),
            pltpu.VMEM_SHARED((npad,), jnp.float32),
            pltpu.SemaphoreType.DMA,
            pltpu.SemaphoreType.DMA,
        ],
    )(_deg_body)
    degp = deg_fn(g4,
                  jnp.ones((chunk,), jnp.float32),
                  jnp.zeros((deg_pt,), jnp.float32))
    deg2t = degp.reshape(2, 2, npad).sum(axis=0).T

    blk = 2000
    grid = (n // blk,)
    z = pl.pallas_call(
        _mm_body,
        grid=grid,
        in_specs=[
            pl.BlockSpec((blk, d), lambda i: (i, 0)),
            pl.BlockSpec((d, d_out), lambda i: (0, 0)),
        ],
        out_specs=pl.BlockSpec((blk, d_out), lambda i: (i, 0)),
        out_shape=jax.ShapeDtypeStruct((n, d_out), jnp.float32),
    )(features, W)

    ys = pl.pallas_call(
        _scale_body,
        grid=grid,
        in_specs=[
            pl.BlockSpec((blk, d_out), lambda i: (i, 0)),
            pl.BlockSpec((blk, 2), lambda i: (i, 0)),
        ],
        out_specs=pl.BlockSpec((blk, d_out), lambda i: (i, 0)),
        out_shape=jax.ShapeDtypeStruct((n, d_out), jnp.float32),
    )(z, deg2t)

    agg_fn = functools.partial(
        pl.kernel,
        out_type=jax.ShapeDtypeStruct((NC, npad, d_out), jnp.float32),
        mesh=mesh,
        scratch_types=(
            [pltpu.VMEM((chunk,), jnp.int32) for _ in range(2 * NBUF)]
            + [pltpu.VMEM((chunk, d_out), jnp.float32) for _ in range(NBUF)]
            + [pltpu.VMEM_SHARED((npad, d_out), jnp.float32)]
            + [pltpu.SemaphoreType.DMA for _ in range(4 * NBUF)]
        ),
    )(_agg_body)
    parts = agg_fn(ys, g4,
                   jnp.zeros((chunk, d_out), jnp.float32))

    out = pl.pallas_call(
        _fin_body,
        grid=grid,
        in_specs=[
            pl.BlockSpec((1, blk, d_out), lambda i: (0, i, 0)),
            pl.BlockSpec((1, blk, d_out), lambda i: (1, i, 0)),
            pl.BlockSpec((blk, 2), lambda i: (i, 0)),
            pl.BlockSpec((1, d_out), lambda i: (0, 0)),
        ],
        out_specs=pl.BlockSpec((blk, d_out), lambda i: (i, 0)),
        out_shape=jax.ShapeDtypeStruct((n, d_out), jnp.float32),
    )(parts, parts, deg2t, b.reshape(1, d_out))

    return (g, out)


# speedup vs baseline: 15.3887x; 1.0008x over previous
import functools

import jax
import jax.numpy as jnp
from jax import lax
from jax.experimental import pallas as pl
from jax.experimental.pallas import tpu as pltpu
from jax.experimental.pallas import tpu_sc as plsc

NC = 2
NS = 16
NW = NC * NS


def _deg_body(g4, ones_h, zeros_h, out, sidx, didx, ones_v, zbuf,
              dbuf, acc_s, acc_d, ss, sd):
    npad = acc_s.shape[0]
    per_tile = npad // NS
    cid = lax.axis_index("c")
    sid = lax.axis_index("s")
    wid = cid * NS + sid
    nch = sidx.shape[0]
    depth = 8

    pltpu.sync_copy(g4.at[0, wid], sidx)
    pltpu.sync_copy(g4.at[1, wid], didx)
    pltpu.sync_copy(ones_h, ones_v)
    pltpu.sync_copy(zeros_h, zbuf)

    base = sid * per_tile
    pltpu.sync_copy(zbuf, acc_s.at[pl.ds(base, per_tile)])
    pltpu.sync_copy(zbuf, acc_d.at[pl.ds(base, per_tile)])
    plsc.subcore_barrier()

    unroll = 5
    ngr = nch // unroll
    assert nch % unroll == 0

    def body(gi, carry):
        for u in range(unroll):
            j = gi * unroll + u
            pltpu.async_copy(ones_v, acc_s.at[sidx.at[j]], ss, add=True)
            pltpu.async_copy(ones_v, acc_d.at[didx.at[j]], sd, add=True)

        @pl.when(gi >= depth)
        def _():
            for _u in range(unroll):
                pltpu.make_async_copy(ones_v, acc_s.at[sidx.at[0]],
                                      ss).wait()
                pltpu.make_async_copy(ones_v, acc_d.at[didx.at[0]],
                                      sd).wait()

        return carry

    lax.fori_loop(0, ngr, body, 0)

    def drain(gi, carry):
        for _u in range(unroll):
            pltpu.make_async_copy(ones_v, acc_s.at[sidx.at[0]], ss).wait()
            pltpu.make_async_copy(ones_v, acc_d.at[didx.at[0]], sd).wait()
        return carry

    lax.fori_loop(0, min(depth, ngr), drain, 0)
    plsc.subcore_barrier()

    pltpu.sync_copy(acc_s.at[pl.ds(base, per_tile)], dbuf)
    pltpu.sync_copy(dbuf, out.at[pl.ds(cid * 2 * npad + base, per_tile)])
    pltpu.sync_copy(acc_d.at[pl.ds(base, per_tile)], dbuf)
    pltpu.sync_copy(dbuf, out.at[pl.ds((cid * 2 + 1) * npad + base,
                                       per_tile)])


NBUF = 4


def _agg_body(ys, g4, zeros_h, out,
              sb0, sb1, sb2, sb3, db0, db1, db2, db3, r0, r1, r2, r3, acc,
              ss0, ss1, ss2, ss3, ds0, ds1, ds2, ds3,
              gs0, gs1, gs2, gs3, cs0, cs1, cs2, cs3):
    npad, dd = acc.shape
    rows_per_tile = npad // NS
    chunk = r0.shape[0]
    nch = g4.shape[2]
    ncopy = rows_per_tile // chunk
    cid = lax.axis_index("c")
    sid = lax.axis_index("s")
    wid = cid * NS + sid

    sb = (sb0, sb1, sb2, sb3)
    db = (db0, db1, db2, db3)
    rows = (r0, r1, r2, r3)
    ssem = (ss0, ss1, ss2, ss3)
    dsem = (ds0, ds1, ds2, ds3)
    gsem = (gs0, gs1, gs2, gs3)
    csem = (cs0, cs1, cs2, cs3)

    pltpu.sync_copy(zeros_h, r0)
    rowbase = sid * rows_per_tile
    for k in range(ncopy):
        sl = pl.ds(rowbase + k * chunk, chunk)
        pltpu.async_copy(r0, acc.at[sl], gs0)
    for k in range(ncopy):
        sl = pl.ds(rowbase + k * chunk, chunk)
        pltpu.make_async_copy(r0, acc.at[sl], gs0).wait()
    plsc.subcore_barrier()

    def _prep(t, j):
        pltpu.async_copy(g4.at[0, wid, j], sb[t], ssem[t])
        pltpu.async_copy(g4.at[1, wid, j], db[t], dsem[t])
        pltpu.make_async_copy(g4.at[0, wid, 0], sb[t], ssem[t]).wait()
        pltpu.async_copy(ys.at[sb[t]], rows[t], gsem[t])

    for b in range(min(3, nch)):
        _prep(b, b)

    def group(gi, carry):
        for b in range(NBUF):
            j = NBUF * gi + b
            s = b
            pltpu.make_async_copy(ys.at[pl.ds(0, chunk)], rows[s],
                                  gsem[s]).wait()
            pltpu.make_async_copy(g4.at[1, wid, 0], db[s], dsem[s]).wait()
            pltpu.async_copy(rows[s], acc.at[db[s]], csem[s], add=True)
            t = (b + 3) % NBUF

            @pl.when(j + 3 < nch)
            def _():
                @pl.when(j >= 1)
                def _():
                    pltpu.make_async_copy(rows[t], acc.at[db[t]],
                                          csem[t]).wait()

                _prep(t, j + 3)

        return carry

    ngroups = (nch - 1) // NBUF
    lax.fori_loop(0, ngroups, group, 0)
    for j in range(ngroups * NBUF, nch):
        s = j % NBUF
        pltpu.make_async_copy(ys.at[pl.ds(0, chunk)], rows[s],
                              gsem[s]).wait()
        pltpu.make_async_copy(g4.at[1, wid, 0], db[s], dsem[s]).wait()
        pltpu.async_copy(rows[s], acc.at[db[s]], csem[s], add=True)
    for j in range(max(nch - NBUF, 0), nch):
        s = j % NBUF
        pltpu.make_async_copy(rows[s], acc.at[db[s]], csem[s]).wait()
    plsc.subcore_barrier()

    for k in range(ncopy):
        b = k % 2
        sl = pl.ds(rowbase + k * chunk, chunk)
        if k >= 2:
            slp = pl.ds(rowbase + (k - 2) * chunk, chunk)
            pltpu.make_async_copy(rows[b], out.at[cid, slp],
                                  gsem[b]).wait()
        pltpu.sync_copy(acc.at[sl], rows[b])
        pltpu.async_copy(rows[b], out.at[cid, sl], gsem[b])
    for k in range(max(ncopy - 2, 0), ncopy):
        b = k % 2
        sl = pl.ds(rowbase + k * chunk, chunk)
        pltpu.make_async_copy(rows[b], out.at[cid, sl], gsem[b]).wait()


def _mm_body(x_ref, w_ref, o_ref):
    o_ref[...] = jnp.dot(x_ref[...], w_ref[...],
                         preferred_element_type=jnp.float32)


def _scale_body(z_ref, deg_ref, o_ref):
    norm = lax.rsqrt(jnp.maximum(deg_ref[:, 0:1], 1.0))
    o_ref[...] = z_ref[...] * norm


def _fin_body(a0_ref, a1_ref, deg_ref, b_ref, o_ref):
    norm = lax.rsqrt(jnp.maximum(deg_ref[:, 1:2], 1.0))
    s = (a0_ref[0] + a1_ref[0]) * norm + b_ref[...]
    o_ref[...] = jnp.maximum(s, 0.0)


def kernel(g, features, W, b):
    n, d = features.shape
    e = g.shape[1]
    d_out = W.shape[1]

    chunk = 80
    assert e % (NW * chunk) == 0
    nch = e // (NW * chunk)
    npad = ((n + NS * 128 - 1) // (NS * 128)) * (NS * 128)
    deg_pt = npad // NS
    assert (npad // NS) % chunk == 0

    g32 = g.astype(jnp.int32)
    g4 = g32.reshape(2, NW, nch, chunk)

    mesh = plsc.VectorSubcoreMesh(core_axis_name="c", subcore_axis_name="s",
                                  num_cores=NC, num_subcores=NS)

    deg_fn = functools.partial(
        pl.kernel,
        out_type=jax.ShapeDtypeStruct((NC * 2 * npad,), jnp.float32),
        mesh=mesh,
        scratch_types=[
            pltpu.VMEM((nch, chunk), jnp.int32),
            pltpu.VMEM((nch, chunk), jnp.int32),
            pltpu.VMEM((chunk,), jnp.float32),
            pltpu.VMEM((deg_pt,), jnp.float32),
            pltpu.VMEM((deg_pt,), jnp.float32),
            pltpu.VMEM_SHARED((npad,), jnp.float32),
            pltpu.VMEM_SHARED((npad,), jnp.float32),
            pltpu.SemaphoreType.DMA,
            pltpu.SemaphoreType.DMA,
        ],
    )(_deg_body)
    degp = deg_fn(g4,
                  jnp.ones((chunk,), jnp.float32),
                  jnp.zeros((deg_pt,), jnp.float32))
    deg2t = degp.reshape(2, 2, npad).sum(axis=0).T

    blk = 2000
    grid = (n // blk,)
    z = pl.pallas_call(
        _mm_body,
        grid=grid,
        in_specs=[
            pl.BlockSpec((blk, d), lambda i: (i, 0)),
            pl.BlockSpec((d, d_out), lambda i: (0, 0)),
        ],
        out_specs=pl.BlockSpec((blk, d_out), lambda i: (i, 0)),
        out_shape=jax.ShapeDtypeStruct((n, d_out), jnp.float32),
    )(features, W)

    ys = pl.pallas_call(
        _scale_body,
        grid=grid,
        in_specs=[
            pl.BlockSpec((blk, d_out), lambda i: (i, 0)),
            pl.BlockSpec((blk, 2), lambda i: (i, 0)),
        ],
        out_specs=pl.BlockSpec((blk, d_out), lambda i: (i, 0)),
        out_shape=jax.ShapeDtypeStruct((n, d_out), jnp.float32),
    )(z, deg2t)

    agg_fn = functools.partial(
        pl.kernel,
        out_type=jax.ShapeDtypeStruct((NC, npad, d_out), jnp.float32),
        mesh=mesh,
        scratch_types=(
            [pltpu.VMEM((chunk,), jnp.int32) for _ in range(2 * NBUF)]
            + [pltpu.VMEM((chunk, d_out), jnp.float32) for _ in range(NBUF)]
            + [pltpu.VMEM_SHARED((npad, d_out), jnp.float32)]
            + [pltpu.SemaphoreType.DMA for _ in range(4 * NBUF)]
        ),
    )(_agg_body)
    parts = agg_fn(ys, g4,
                   jnp.zeros((chunk, d_out), jnp.float32))

    out = pl.pallas_call(
        _fin_body,
        grid=grid,
        in_specs=[
            pl.BlockSpec((1, blk, d_out), lambda i: (0, i, 0)),
            pl.BlockSpec((1, blk, d_out), lambda i: (1, i, 0)),
            pl.BlockSpec((blk, 2), lambda i: (i, 0)),
            pl.BlockSpec((1, d_out), lambda i: (0, 0)),
        ],
        out_specs=pl.BlockSpec((blk, d_out), lambda i: (i, 0)),
        out_shape=jax.ShapeDtypeStruct((n, d_out), jnp.float32),
    )(parts, parts, deg2t, b.reshape(1, d_out))

    return (g4.reshape(2, e), out)
